# speedup vs baseline: 91.6931x; 91.6931x over previous
import jax
import jax.numpy as jnp
from jax.experimental import pallas as pl

B, N, ATOM_DIM, EDGE_DIM, HID, HEADS = 256, 32, 128, 16, 75, 8
GB = 8


def _masked_softmax_over_i(logits, mask):
    neg = jnp.float32(-1e30)
    ml = jnp.where(mask, logits, neg)
    mx = jnp.max(ml, axis=1, keepdims=True)
    ex = jnp.where(mask, jnp.exp(logits - mx), 0.0)
    den = jnp.sum(ex, axis=1, keepdims=True)
    return ex / (den + 1e-16)


def _gat_kernel(atoms_ref, adjs_ref, edges_ref,
                w1_ref, as1_ref, ad1_ref, le1_ref, ae1_ref, b1_ref,
                w2_ref, as2_ref, ad2_ref, le2_ref, ae2_ref, b2_ref,
                out_ref):
    x = atoms_ref[...].reshape(GB * N, ATOM_DIM)
    mask = adjs_ref[...] > 0.5
    ef = edges_ref[...].reshape(GB * N * N, EDGE_DIM)

    h1 = jnp.dot(x, w1_ref[...], preferred_element_type=jnp.float32)

    x1_cols = []
    for h in range(HEADS):
        sl = slice(h * HID, (h + 1) * HID)
        h1h = h1[:, sl]
        a_s = h1h @ as1_ref[h, :]
        a_d = h1h @ ad1_ref[h, :]
        ve = le1_ref[:, sl] @ ae1_ref[h, :]
        a_e = (ef @ ve).reshape(GB, N, N)
        logits = (a_s.reshape(GB, N, 1) + a_d.reshape(GB, 1, N) + a_e)
        logits = jnp.where(logits >= 0, logits, 0.2 * logits)
        alpha = _masked_softmax_over_i(logits, mask)
        outh = jax.lax.dot_general(
            alpha, h1h.reshape(GB, N, HID),
            (((1,), (1,)), ((0,), (0,))),
            preferred_element_type=jnp.float32)
        x1_cols.append(outh.reshape(GB * N, HID))
    x1 = jnp.concatenate(x1_cols, axis=1) + b1_ref[...]

    h2 = jnp.dot(x1, w2_ref[...], preferred_element_type=jnp.float32)
    a_s = h2 @ as2_ref[0, :]
    a_d = h2 @ ad2_ref[0, :]
    ve2 = le2_ref[...] @ ae2_ref[0, :]
    a_e = (ef @ ve2).reshape(GB, N, N)
    logits = (a_s.reshape(GB, N, 1) + a_d.reshape(GB, 1, N) + a_e)
    logits = jnp.where(logits >= 0, logits, 0.2 * logits)
    alpha = _masked_softmax_over_i(logits, mask)
    out = jax.lax.dot_general(
        alpha, h2.reshape(GB, N, HID),
        (((1,), (1,)), ((0,), (0,))),
        preferred_element_type=jnp.float32)
    out_ref[...] = out + b2_ref[...]


@jax.jit
def kernel(atoms, adjs, edges, W1, att_src1, att_dst1, lin_e1, att_e1, b1,
           W2, att_src2, att_dst2, lin_e2, att_e2, b2):
    grid = (B // GB,)
    bcast = lambda shape: pl.BlockSpec(shape, lambda g: (0,) * len(shape))
    out = pl.pallas_call(
        _gat_kernel,
        grid=grid,
        in_specs=[
            pl.BlockSpec((GB, N, ATOM_DIM), lambda g: (g, 0, 0)),
            pl.BlockSpec((GB, N, N), lambda g: (g, 0, 0)),
            pl.BlockSpec((GB, N, N, EDGE_DIM), lambda g: (g, 0, 0, 0)),
            bcast((ATOM_DIM, HEADS * HID)),
            bcast((HEADS, HID)),
            bcast((HEADS, HID)),
            bcast((EDGE_DIM, HEADS * HID)),
            bcast((HEADS, HID)),
            bcast((HEADS * HID,)),
            bcast((HEADS * HID, HID)),
            bcast((1, HID)),
            bcast((1, HID)),
            bcast((EDGE_DIM, HID)),
            bcast((1, HID)),
            bcast((HID,)),
        ],
        out_specs=pl.BlockSpec((GB, N, HID), lambda g: (g, 0, 0)),
        out_shape=jax.ShapeDtypeStruct((B, N, HID), jnp.float32),
    )(atoms, adjs, edges, W1, att_src1, att_dst1, lin_e1, att_e1, b1,
      W2, att_src2, att_dst2, lin_e2, att_e2, b2)
    return out


# speedup vs baseline: 334.0999x; 3.6437x over previous
import jax
import jax.numpy as jnp
from jax import lax
from jax.experimental import pallas as pl

B, N, ATOM_DIM, EDGE_DIM, HID, HEADS = 256, 32, 128, 16, 75, 8
GB = 8
R = GB * N

_CT = (((1,), (1,)), ((), ()))
_C0 = (((0,), (0,)), ((), ()))


def _masked_softmax_cols(logits, mask):
    ml = jnp.where(mask, logits, jnp.float32(-1e30))
    mx = jnp.max(ml, axis=0, keepdims=True)
    ex = jnp.where(mask, jnp.exp(logits - mx), 0.0)
    den = jnp.sum(ex, axis=0, keepdims=True)
    return ex / (den + 1e-16)


def _leaky(x):
    return jnp.where(x >= 0, x, 0.2 * x)


def _edge_weight_mat(ve, heads):
    w = jnp.broadcast_to(ve[:, :, None], (EDGE_DIM, heads, N))
    w = w.reshape(EDGE_DIM, heads * N)
    w = jnp.broadcast_to(w[None, :, :], (N, EDGE_DIM, heads * N))
    w = w.reshape(N * EDGE_DIM, heads * N)
    p = lax.broadcasted_iota(jnp.int32, (N * EDGE_DIM, heads * N), 0)
    q = lax.broadcasted_iota(jnp.int32, (N * EDGE_DIM, heads * N), 1)
    return w * ((p // EDGE_DIM) == (q % N)).astype(jnp.float32)


def _gat_kernel(atoms_ref, adjs_ref, edges_ref,
                w1_ref, as1_ref, ad1_ref, le1_ref, ae1_ref, b1_ref,
                w2_ref, as2_ref, ad2_ref, le2_ref, ae2_ref, b2_ref,
                out_ref):
    f32 = jnp.float32
    x = atoms_ref[...].reshape(R, ATOM_DIM)
    adj = adjs_ref[...].reshape(R, N)
    er = edges_ref[...].reshape(R, N * EDGE_DIM)

    row_g = lax.broadcasted_iota(jnp.int32, (R, R), 0) // N
    col_g = lax.broadcasted_iota(jnp.int32, (R, R), 1) // N
    mask = (row_g == col_g) & (jnp.tile(adj, (1, GB)) > 0.5)

    kk = lax.broadcasted_iota(jnp.int32, (HEADS, HEADS * HID), 1) // HID
    hh = lax.broadcasted_iota(jnp.int32, (HEADS, HEADS * HID), 0)
    hmask = (kk == hh).astype(f32)
    as1_blk = jnp.tile(as1_ref[...], (1, HEADS)) * hmask
    ad1_blk = jnp.tile(ad1_ref[...], (1, HEADS)) * hmask
    ae1_blk = jnp.tile(ae1_ref[...], (1, HEADS)) * hmask

    h1 = jnp.dot(x, w1_ref[...], preferred_element_type=f32)
    a_s = lax.dot_general(h1, as1_blk, _CT,
                          preferred_element_type=f32)
    a_d = lax.dot_general(ad1_blk, h1, _CT,
                          preferred_element_type=f32)
    ve1 = lax.dot_general(le1_ref[...], ae1_blk, _CT,
                          preferred_element_type=f32)
    ae_mat = _edge_weight_mat(ve1, HEADS)
    a_e = jnp.dot(er, ae_mat, preferred_element_type=f32)

    x1_cols = []
    for h in range(HEADS):
        lg = (a_s[:, h][:, None] + a_d[h, :][None, :]
              + jnp.tile(a_e[:, h * N:(h + 1) * N], (1, GB)))
        alpha = _masked_softmax_cols(_leaky(lg), mask)
        x1_cols.append(lax.dot_general(
            alpha, h1[:, h * HID:(h + 1) * HID], _C0,
            preferred_element_type=f32))
    x1 = jnp.concatenate(x1_cols, axis=1) + b1_ref[...]

    h2 = jnp.dot(x1, w2_ref[...], preferred_element_type=f32)
    a_s2 = lax.dot_general(h2, as2_ref[...], _CT,
                           preferred_element_type=f32)
    a_d2 = lax.dot_general(ad2_ref[...], h2, _CT,
                           preferred_element_type=f32)
    ve2 = lax.dot_general(le2_ref[...], ae2_ref[...], _CT,
                          preferred_element_type=f32)
    ae2_mat = _edge_weight_mat(ve2, 1)
    a_e2 = jnp.dot(er, ae2_mat, preferred_element_type=f32)

    lg2 = a_s2 + a_d2 + jnp.tile(a_e2, (1, GB))
    alpha2 = _masked_softmax_cols(_leaky(lg2), mask)
    out = lax.dot_general(alpha2, h2, _C0,
                          preferred_element_type=f32)
    out_ref[...] = (out + b2_ref[...]).reshape(GB, N, HID)


@jax.jit
def kernel(atoms, adjs, edges, W1, att_src1, att_dst1, lin_e1, att_e1, b1,
           W2, att_src2, att_dst2, lin_e2, att_e2, b2):
    grid = (B // GB,)
    bcast = lambda shape: pl.BlockSpec(shape, lambda g: (0,) * len(shape))
    out = pl.pallas_call(
        _gat_kernel,
        grid=grid,
        in_specs=[
            pl.BlockSpec((GB, N, ATOM_DIM), lambda g: (g, 0, 0)),
            pl.BlockSpec((GB, N, N), lambda g: (g, 0, 0)),
            pl.BlockSpec((GB, N, N * EDGE_DIM), lambda g: (g, 0, 0)),
            bcast((ATOM_DIM, HEADS * HID)),
            bcast((HEADS, HID)),
            bcast((HEADS, HID)),
            bcast((EDGE_DIM, HEADS * HID)),
            bcast((HEADS, HID)),
            bcast((HEADS * HID,)),
            bcast((HEADS * HID, HID)),
            bcast((1, HID)),
            bcast((1, HID)),
            bcast((EDGE_DIM, HID)),
            bcast((1, HID)),
            bcast((HID,)),
        ],
        out_specs=pl.BlockSpec((GB, N, HID), lambda g: (g, 0, 0)),
        out_shape=jax.ShapeDtypeStruct((B, N, HID), jnp.float32),
    )(atoms, adjs, edges.reshape(B, N, N * EDGE_DIM),
      W1, att_src1, att_dst1, lin_e1, att_e1, b1,
      W2, att_src2, att_dst2, lin_e2, att_e2, b2)
    return out


# speedup vs baseline: 391.6891x; 1.1724x over previous
import jax
import jax.numpy as jnp
from jax import lax
from jax.experimental import pallas as pl
from jax.experimental.pallas import tpu as pltpu

B, N, ATOM_DIM, EDGE_DIM, HID, HEADS = 256, 32, 128, 16, 75, 8
GB = 8
R = GB * N

_CT = (((1,), (1,)), ((), ()))
_BAT = (((1,), (1,)), ((0,), (0,)))


def _leaky(x):
    return jnp.maximum(x, 0.2 * x)


def _edge_weight_mat(ve, heads):
    w = jnp.broadcast_to(ve[:, :, None], (EDGE_DIM, heads, N))
    w = w.reshape(EDGE_DIM, heads * N)
    w = jnp.broadcast_to(w[None, :, :], (N, EDGE_DIM, heads * N))
    w = w.reshape(N * EDGE_DIM, heads * N)
    p = lax.broadcasted_iota(jnp.int32, (N * EDGE_DIM, heads * N), 0)
    q = lax.broadcasted_iota(jnp.int32, (N * EDGE_DIM, heads * N), 1)
    return w * ((p // EDGE_DIM) == (q % N)).astype(jnp.float32)


def _hblk(a_ref):
    kk = lax.broadcasted_iota(jnp.int32, (HEADS, HEADS * HID), 1) // HID
    hh = lax.broadcasted_iota(jnp.int32, (HEADS, HEADS * HID), 0)
    return jnp.tile(a_ref[...], (1, HEADS)) * (kk == hh).astype(jnp.float32)


def _gat_kernel(atoms_ref, adjs_ref, edges_ref,
                w1_ref, as1_ref, ad1_ref, le1_ref, ae1_ref, b1_ref,
                w2_ref, as2_ref, ad2_ref, le2_ref, ae2_ref, b2_ref,
                out_ref, ae_mat_ref, ae2_mat_ref):
    f32 = jnp.float32

    @pl.when(pl.program_id(0) == 0)
    def _init_scratch():
        ve1 = lax.dot_general(le1_ref[...], _hblk(ae1_ref), _CT,
                              preferred_element_type=f32)
        ae_mat_ref[...] = _edge_weight_mat(ve1, HEADS)
        ve2 = lax.dot_general(le2_ref[...], ae2_ref[...], _CT,
                              preferred_element_type=f32)
        ae2_mat_ref[...] = _edge_weight_mat(ve2, 1)

    x = atoms_ref[...].reshape(R, ATOM_DIM)
    er = edges_ref[...].reshape(R, N * EDGE_DIM)
    adjbias = jnp.where(adjs_ref[...] > 0.5, 0.0, -1e30)

    rr = lax.broadcasted_iota(jnp.int32, (N, N), 0)
    cc = lax.broadcasted_iota(jnp.int32, (N, N), 1)
    eye = (rr == cc).astype(f32)

    h1 = jnp.dot(x, w1_ref[...], preferred_element_type=f32)
    a_s = lax.dot_general(h1, _hblk(as1_ref), _CT,
                          preferred_element_type=f32)
    a_dc = lax.dot_general(h1, _hblk(ad1_ref), _CT,
                           preferred_element_type=f32)
    a_dt = lax.dot_general(a_dc.reshape(GB, N, HEADS), eye,
                           (((1,), (0,)), ((), ())),
                           preferred_element_type=f32)
    a_e = jnp.dot(er, ae_mat_ref[...],
                  preferred_element_type=f32)

    h13 = h1.reshape(GB, N, HEADS * HID)
    x1_cols = []
    for h in range(HEADS):
        lg = (a_e[:, h * N:(h + 1) * N] + a_s[:, h:h + 1]).reshape(GB, N, N)
        lg = _leaky(lg + a_dt[:, h:h + 1, :])
        ex = jnp.exp(lg + adjbias)
        den = jnp.sum(ex, axis=1, keepdims=True)
        alpha = ex / (den + 1e-16)
        x1_cols.append(lax.dot_general(
            alpha, h13[:, :, h * HID:(h + 1) * HID], _BAT,
            preferred_element_type=f32).reshape(R, HID))
    x1 = jnp.concatenate(x1_cols, axis=1) + b1_ref[...]

    h2 = jnp.dot(x1, w2_ref[...], preferred_element_type=f32)
    a_s2 = lax.dot_general(h2, as2_ref[...], _CT,
                           preferred_element_type=f32)
    a_d2 = lax.dot_general(h2, ad2_ref[...], _CT,
                           preferred_element_type=f32)
    a_d2t = lax.dot_general(a_d2.reshape(GB, N, 1), eye,
                            (((1,), (0,)), ((), ())),
                            preferred_element_type=f32)
    a_e2 = jnp.dot(er, ae2_mat_ref[...],
                   preferred_element_type=f32)

    lg2 = (a_e2 + a_s2).reshape(GB, N, N)
    lg2 = _leaky(lg2 + a_d2t)
    ex2 = jnp.exp(lg2 + adjbias)
    den2 = jnp.sum(ex2, axis=1, keepdims=True)
    alpha2 = ex2 / (den2 + 1e-16)
    out = lax.dot_general(alpha2, h2.reshape(GB, N, HID), _BAT,
                          preferred_element_type=f32)
    out_ref[...] = out + b2_ref[...]


@jax.jit
def kernel(atoms, adjs, edges, W1, att_src1, att_dst1, lin_e1, att_e1, b1,
           W2, att_src2, att_dst2, lin_e2, att_e2, b2):
    grid = (B // GB,)
    bcast = lambda shape: pl.BlockSpec(shape, lambda g: (0,) * len(shape))
    out = pl.pallas_call(
        _gat_kernel,
        grid=grid,
        in_specs=[
            pl.BlockSpec((GB, N, ATOM_DIM), lambda g: (g, 0, 0)),
            pl.BlockSpec((GB, N, N), lambda g: (g, 0, 0)),
            pl.BlockSpec((GB, N, N * EDGE_DIM), lambda g: (g, 0, 0)),
            bcast((ATOM_DIM, HEADS * HID)),
            bcast((HEADS, HID)),
            bcast((HEADS, HID)),
            bcast((EDGE_DIM, HEADS * HID)),
            bcast((HEADS, HID)),
            bcast((HEADS * HID,)),
            bcast((HEADS * HID, HID)),
            bcast((1, HID)),
            bcast((1, HID)),
            bcast((EDGE_DIM, HID)),
            bcast((1, HID)),
            bcast((HID,)),
        ],
        out_specs=pl.BlockSpec((GB, N, HID), lambda g: (g, 0, 0)),
        out_shape=jax.ShapeDtypeStruct((B, N, HID), jnp.float32),
        scratch_shapes=[
            pltpu.VMEM((N * EDGE_DIM, HEADS * N), jnp.float32),
            pltpu.VMEM((N * EDGE_DIM, N), jnp.float32),
        ],
    )(atoms, adjs, edges.reshape(B, N, N * EDGE_DIM),
      W1, att_src1, att_dst1, lin_e1, att_e1, b1,
      W2, att_src2, att_dst2, lin_e2, att_e2, b2)
    return out


# speedup vs baseline: 451.7118x; 1.1532x over previous
import jax
import jax.numpy as jnp
from jax import lax
from jax.experimental import pallas as pl
from jax.experimental.pallas import tpu as pltpu

B, N, ATOM_DIM, EDGE_DIM, HID, HEADS = 256, 32, 128, 16, 75, 8
GB = 16
R = GB * N

_CT = (((1,), (1,)), ((), ()))
_BAT = (((1,), (1,)), ((0,), (0,)))


def _leaky(x):
    return jnp.maximum(x, 0.2 * x)


def _edge_weight_mat(ve, heads):
    w = jnp.broadcast_to(ve[:, :, None], (EDGE_DIM, heads, N))
    w = w.reshape(EDGE_DIM, heads * N)
    w = jnp.broadcast_to(w[None, :, :], (N, EDGE_DIM, heads * N))
    w = w.reshape(N * EDGE_DIM, heads * N)
    p = lax.broadcasted_iota(jnp.int32, (N * EDGE_DIM, heads * N), 0)
    q = lax.broadcasted_iota(jnp.int32, (N * EDGE_DIM, heads * N), 1)
    return w * ((p // EDGE_DIM) == (q % N)).astype(jnp.float32)


def _hblk(a_ref):
    kk = lax.broadcasted_iota(jnp.int32, (HEADS, HEADS * HID), 1) // HID
    hh = lax.broadcasted_iota(jnp.int32, (HEADS, HEADS * HID), 0)
    return jnp.tile(a_ref[...], (1, HEADS)) * (kk == hh).astype(jnp.float32)


def _gat_kernel(atoms_ref, adjs_ref, edges_ref,
                w1_ref, as1_ref, ad1_ref, le1_ref, ae1_ref, b1_ref,
                w2_ref, as2_ref, ad2_ref, le2_ref, ae2_ref, b2_ref,
                out_ref, ae_mat_ref, ae2_mat_ref):
    f32 = jnp.float32

    @pl.when(pl.program_id(0) == 0)
    def _init_scratch():
        ve1 = lax.dot_general(le1_ref[...], _hblk(ae1_ref), _CT,
                              preferred_element_type=f32)
        ae_mat_ref[...] = _edge_weight_mat(ve1, HEADS)
        ve2 = lax.dot_general(le2_ref[...], ae2_ref[...], _CT,
                              preferred_element_type=f32)
        ae2_mat_ref[...] = _edge_weight_mat(ve2, 1)

    x = atoms_ref[...].reshape(R, ATOM_DIM)
    er = edges_ref[...].reshape(R, N * EDGE_DIM)
    adjbias = jnp.where(adjs_ref[...] > 0.5, 0.0, -1e30)

    rr = lax.broadcasted_iota(jnp.int32, (N, N), 0)
    cc = lax.broadcasted_iota(jnp.int32, (N, N), 1)
    eye = (rr == cc).astype(f32)

    h1 = jnp.dot(x, w1_ref[...], preferred_element_type=f32)
    a_s = lax.dot_general(h1, _hblk(as1_ref), _CT,
                          preferred_element_type=f32)
    a_dc = lax.dot_general(h1, _hblk(ad1_ref), _CT,
                           preferred_element_type=f32)
    a_dt = lax.dot_general(a_dc.reshape(GB, N, HEADS), eye,
                           (((1,), (0,)), ((), ())),
                           preferred_element_type=f32)
    a_e = jnp.dot(er, ae_mat_ref[...],
                  preferred_element_type=f32)

    h13 = h1.reshape(GB, N, HEADS * HID)
    x1_cols = []
    for h in range(HEADS):
        lg = (a_e[:, h * N:(h + 1) * N] + a_s[:, h:h + 1]).reshape(GB, N, N)
        lg = _leaky(lg + a_dt[:, h:h + 1, :])
        ex = jnp.exp(lg + adjbias)
        den = jnp.sum(ex, axis=1, keepdims=True)
        alpha = ex / (den + 1e-16)
        x1_cols.append(lax.dot_general(
            alpha, h13[:, :, h * HID:(h + 1) * HID], _BAT,
            preferred_element_type=f32).reshape(R, HID))
    x1 = jnp.concatenate(x1_cols, axis=1) + b1_ref[...]

    h2 = jnp.dot(x1, w2_ref[...], preferred_element_type=f32)
    a_s2 = lax.dot_general(h2, as2_ref[...], _CT,
                           preferred_element_type=f32)
    a_d2 = lax.dot_general(h2, ad2_ref[...], _CT,
                           preferred_element_type=f32)
    a_d2t = lax.dot_general(a_d2.reshape(GB, N, 1), eye,
                            (((1,), (0,)), ((), ())),
                            preferred_element_type=f32)
    a_e2 = jnp.dot(er, ae2_mat_ref[...],
                   preferred_element_type=f32)

    lg2 = (a_e2 + a_s2).reshape(GB, N, N)
    lg2 = _leaky(lg2 + a_d2t)
    ex2 = jnp.exp(lg2 + adjbias)
    den2 = jnp.sum(ex2, axis=1, keepdims=True)
    alpha2 = ex2 / (den2 + 1e-16)
    out = lax.dot_general(alpha2, h2.reshape(GB, N, HID), _BAT,
                          preferred_element_type=f32)
    out_ref[...] = out + b2_ref[...]


@jax.jit
def kernel(atoms, adjs, edges, W1, att_src1, att_dst1, lin_e1, att_e1, b1,
           W2, att_src2, att_dst2, lin_e2, att_e2, b2):
    grid = (B // GB,)
    bcast = lambda shape: pl.BlockSpec(shape, lambda g: (0,) * len(shape))
    out = pl.pallas_call(
        _gat_kernel,
        grid=grid,
        in_specs=[
            pl.BlockSpec((GB, N, ATOM_DIM), lambda g: (g, 0, 0)),
            pl.BlockSpec((GB, N, N), lambda g: (g, 0, 0)),
            pl.BlockSpec((GB, N, N * EDGE_DIM), lambda g: (g, 0, 0)),
            bcast((ATOM_DIM, HEADS * HID)),
            bcast((HEADS, HID)),
            bcast((HEADS, HID)),
            bcast((EDGE_DIM, HEADS * HID)),
            bcast((HEADS, HID)),
            bcast((HEADS * HID,)),
            bcast((HEADS * HID, HID)),
            bcast((1, HID)),
            bcast((1, HID)),
            bcast((EDGE_DIM, HID)),
            bcast((1, HID)),
            bcast((HID,)),
        ],
        out_specs=pl.BlockSpec((GB, N, HID), lambda g: (g, 0, 0)),
        out_shape=jax.ShapeDtypeStruct((B, N, HID), jnp.float32),
        scratch_shapes=[
            pltpu.VMEM((N * EDGE_DIM, HEADS * N), jnp.float32),
            pltpu.VMEM((N * EDGE_DIM, N), jnp.float32),
        ],
    )(atoms, adjs, edges.reshape(B, N, N * EDGE_DIM),
      W1, att_src1, att_dst1, lin_e1, att_e1, b1,
      W2, att_src2, att_dst2, lin_e2, att_e2, b2)
    return out


# speedup vs baseline: 507.5986x; 1.1237x over previous
import jax
import jax.numpy as jnp
from jax import lax
from jax.experimental import pallas as pl
from jax.experimental.pallas import tpu as pltpu

B, N, ATOM_DIM, EDGE_DIM, HID, HEADS = 256, 32, 128, 16, 75, 8
GB = 32
R = GB * N

_CT = (((1,), (1,)), ((), ()))
_BAT = (((1,), (1,)), ((0,), (0,)))


def _leaky(x):
    return jnp.maximum(x, 0.2 * x)


def _edge_weight_mat(ve, heads):
    w = jnp.broadcast_to(ve[:, :, None], (EDGE_DIM, heads, N))
    w = w.reshape(EDGE_DIM, heads * N)
    w = jnp.broadcast_to(w[None, :, :], (N, EDGE_DIM, heads * N))
    w = w.reshape(N * EDGE_DIM, heads * N)
    p = lax.broadcasted_iota(jnp.int32, (N * EDGE_DIM, heads * N), 0)
    q = lax.broadcasted_iota(jnp.int32, (N * EDGE_DIM, heads * N), 1)
    return w * ((p // EDGE_DIM) == (q % N)).astype(jnp.float32)


def _hblk(a_ref):
    kk = lax.broadcasted_iota(jnp.int32, (HEADS, HEADS * HID), 1) // HID
    hh = lax.broadcasted_iota(jnp.int32, (HEADS, HEADS * HID), 0)
    return jnp.tile(a_ref[...], (1, HEADS)) * (kk == hh).astype(jnp.float32)


def _gat_kernel(atoms_ref, adjs_ref, edges_ref,
                w1_ref, as1_ref, ad1_ref, le1_ref, ae1_ref, b1_ref,
                w2_ref, as2_ref, ad2_ref, le2_ref, ae2_ref, b2_ref,
                out_ref, ae_mat_ref, ae2_mat_ref):
    f32 = jnp.float32

    @pl.when(pl.program_id(0) == 0)
    def _init_scratch():
        ve1 = lax.dot_general(le1_ref[...], _hblk(ae1_ref), _CT,
                              preferred_element_type=f32)
        ae_mat_ref[...] = _edge_weight_mat(ve1, HEADS)
        ve2 = lax.dot_general(le2_ref[...], ae2_ref[...], _CT,
                              preferred_element_type=f32)
        ae2_mat_ref[...] = _edge_weight_mat(ve2, 1)

    x = atoms_ref[...].reshape(R, ATOM_DIM)
    er = edges_ref[...].reshape(R, N * EDGE_DIM)
    adjbias = jnp.where(adjs_ref[...] > 0.5, 0.0, -1e30)

    rr = lax.broadcasted_iota(jnp.int32, (N, N), 0)
    cc = lax.broadcasted_iota(jnp.int32, (N, N), 1)
    eye = (rr == cc).astype(f32)

    h1 = jnp.dot(x, w1_ref[...], preferred_element_type=f32)
    a_s = lax.dot_general(h1, _hblk(as1_ref), _CT,
                          preferred_element_type=f32)
    a_dc = lax.dot_general(h1, _hblk(ad1_ref), _CT,
                           preferred_element_type=f32)
    a_dt = lax.dot_general(a_dc.reshape(GB, N, HEADS), eye,
                           (((1,), (0,)), ((), ())),
                           preferred_element_type=f32)
    a_e = jnp.dot(er, ae_mat_ref[...],
                  preferred_element_type=f32)

    h13 = h1.reshape(GB, N, HEADS * HID)
    x1_cols = []
    for h in range(HEADS):
        lg = (a_e[:, h * N:(h + 1) * N] + a_s[:, h:h + 1]).reshape(GB, N, N)
        lg = _leaky(lg + a_dt[:, h:h + 1, :])
        ex = jnp.exp(lg + adjbias)
        den = jnp.sum(ex, axis=1, keepdims=True)
        alpha = ex / (den + 1e-16)
        x1_cols.append(lax.dot_general(
            alpha, h13[:, :, h * HID:(h + 1) * HID], _BAT,
            preferred_element_type=f32).reshape(R, HID))
    x1 = jnp.concatenate(x1_cols, axis=1) + b1_ref[...]

    h2 = jnp.dot(x1, w2_ref[...], preferred_element_type=f32)
    a_s2 = lax.dot_general(h2, as2_ref[...], _CT,
                           preferred_element_type=f32)
    a_d2 = lax.dot_general(h2, ad2_ref[...], _CT,
                           preferred_element_type=f32)
    a_d2t = lax.dot_general(a_d2.reshape(GB, N, 1), eye,
                            (((1,), (0,)), ((), ())),
                            preferred_element_type=f32)
    a_e2 = jnp.dot(er, ae2_mat_ref[...],
                   preferred_element_type=f32)

    lg2 = (a_e2 + a_s2).reshape(GB, N, N)
    lg2 = _leaky(lg2 + a_d2t)
    ex2 = jnp.exp(lg2 + adjbias)
    den2 = jnp.sum(ex2, axis=1, keepdims=True)
    alpha2 = ex2 / (den2 + 1e-16)
    out = lax.dot_general(alpha2, h2.reshape(GB, N, HID), _BAT,
                          preferred_element_type=f32)
    out_ref[...] = out + b2_ref[...]


@jax.jit
def kernel(atoms, adjs, edges, W1, att_src1, att_dst1, lin_e1, att_e1, b1,
           W2, att_src2, att_dst2, lin_e2, att_e2, b2):
    grid = (B // GB,)
    bcast = lambda shape: pl.BlockSpec(shape, lambda g: (0,) * len(shape))
    out = pl.pallas_call(
        _gat_kernel,
        grid=grid,
        in_specs=[
            pl.BlockSpec((GB, N, ATOM_DIM), lambda g: (g, 0, 0)),
            pl.BlockSpec((GB, N, N), lambda g: (g, 0, 0)),
            pl.BlockSpec((GB, N, N * EDGE_DIM), lambda g: (g, 0, 0)),
            bcast((ATOM_DIM, HEADS * HID)),
            bcast((HEADS, HID)),
            bcast((HEADS, HID)),
            bcast((EDGE_DIM, HEADS * HID)),
            bcast((HEADS, HID)),
            bcast((HEADS * HID,)),
            bcast((HEADS * HID, HID)),
            bcast((1, HID)),
            bcast((1, HID)),
            bcast((EDGE_DIM, HID)),
            bcast((1, HID)),
            bcast((HID,)),
        ],
        out_specs=pl.BlockSpec((GB, N, HID), lambda g: (g, 0, 0)),
        out_shape=jax.ShapeDtypeStruct((B, N, HID), jnp.float32),
        scratch_shapes=[
            pltpu.VMEM((N * EDGE_DIM, HEADS * N), jnp.float32),
            pltpu.VMEM((N * EDGE_DIM, N), jnp.float32),
        ],
    )(atoms, adjs, edges.reshape(B, N, N * EDGE_DIM),
      W1, att_src1, att_dst1, lin_e1, att_e1, b1,
      W2, att_src2, att_dst2, lin_e2, att_e2, b2)
    return out


# speedup vs baseline: 512.4772x; 1.0096x over previous
import jax
import jax.numpy as jnp
from jax import lax
from jax.experimental import pallas as pl
from jax.experimental.pallas import tpu as pltpu

B, N, ATOM_DIM, EDGE_DIM, HID, HEADS = 256, 32, 128, 16, 75, 8
GB = 64
R = GB * N

_CT = (((1,), (1,)), ((), ()))
_BAT = (((1,), (1,)), ((0,), (0,)))


def _leaky(x):
    return jnp.maximum(x, 0.2 * x)


def _edge_weight_mat(ve, heads):
    w = jnp.broadcast_to(ve[:, :, None], (EDGE_DIM, heads, N))
    w = w.reshape(EDGE_DIM, heads * N)
    w = jnp.broadcast_to(w[None, :, :], (N, EDGE_DIM, heads * N))
    w = w.reshape(N * EDGE_DIM, heads * N)
    p = lax.broadcasted_iota(jnp.int32, (N * EDGE_DIM, heads * N), 0)
    q = lax.broadcasted_iota(jnp.int32, (N * EDGE_DIM, heads * N), 1)
    return w * ((p // EDGE_DIM) == (q % N)).astype(jnp.float32)


def _hblk(a_ref):
    kk = lax.broadcasted_iota(jnp.int32, (HEADS, HEADS * HID), 1) // HID
    hh = lax.broadcasted_iota(jnp.int32, (HEADS, HEADS * HID), 0)
    return jnp.tile(a_ref[...], (1, HEADS)) * (kk == hh).astype(jnp.float32)


def _gat_kernel(atoms_ref, adjs_ref, edges_ref,
                w1_ref, as1_ref, ad1_ref, le1_ref, ae1_ref, b1_ref,
                w2_ref, as2_ref, ad2_ref, le2_ref, ae2_ref, b2_ref,
                out_ref, ae_mat_ref, ae2_mat_ref):
    f32 = jnp.float32

    @pl.when(pl.program_id(0) == 0)
    def _init_scratch():
        ve1 = lax.dot_general(le1_ref[...], _hblk(ae1_ref), _CT,
                              preferred_element_type=f32)
        ae_mat_ref[...] = _edge_weight_mat(ve1, HEADS)
        ve2 = lax.dot_general(le2_ref[...], ae2_ref[...], _CT,
                              preferred_element_type=f32)
        ae2_mat_ref[...] = _edge_weight_mat(ve2, 1)

    x = atoms_ref[...].reshape(R, ATOM_DIM)
    er = edges_ref[...].reshape(R, N * EDGE_DIM)
    adjbias = jnp.where(adjs_ref[...] > 0.5, 0.0, -1e30)

    rr = lax.broadcasted_iota(jnp.int32, (N, N), 0)
    cc = lax.broadcasted_iota(jnp.int32, (N, N), 1)
    eye = (rr == cc).astype(f32)

    h1 = jnp.dot(x, w1_ref[...], preferred_element_type=f32)
    a_s = lax.dot_general(h1, _hblk(as1_ref), _CT,
                          preferred_element_type=f32)
    a_dc = lax.dot_general(h1, _hblk(ad1_ref), _CT,
                           preferred_element_type=f32)
    a_dt = lax.dot_general(a_dc.reshape(GB, N, HEADS), eye,
                           (((1,), (0,)), ((), ())),
                           preferred_element_type=f32)
    a_e = jnp.dot(er, ae_mat_ref[...],
                  preferred_element_type=f32)

    h13 = h1.reshape(GB, N, HEADS * HID)
    x1_cols = []
    for h in range(HEADS):
        lg = (a_e[:, h * N:(h + 1) * N] + a_s[:, h:h + 1]).reshape(GB, N, N)
        lg = _leaky(lg + a_dt[:, h:h + 1, :])
        ex = jnp.exp(lg + adjbias)
        den = jnp.sum(ex, axis=1, keepdims=True)
        alpha = ex / (den + 1e-16)
        x1_cols.append(lax.dot_general(
            alpha, h13[:, :, h * HID:(h + 1) * HID], _BAT,
            preferred_element_type=f32).reshape(R, HID))
    x1 = jnp.concatenate(x1_cols, axis=1) + b1_ref[...]

    h2 = jnp.dot(x1, w2_ref[...], preferred_element_type=f32)
    a_s2 = lax.dot_general(h2, as2_ref[...], _CT,
                           preferred_element_type=f32)
    a_d2 = lax.dot_general(h2, ad2_ref[...], _CT,
                           preferred_element_type=f32)
    a_d2t = lax.dot_general(a_d2.reshape(GB, N, 1), eye,
                            (((1,), (0,)), ((), ())),
                            preferred_element_type=f32)
    a_e2 = jnp.dot(er, ae2_mat_ref[...],
                   preferred_element_type=f32)

    lg2 = (a_e2 + a_s2).reshape(GB, N, N)
    lg2 = _leaky(lg2 + a_d2t)
    ex2 = jnp.exp(lg2 + adjbias)
    den2 = jnp.sum(ex2, axis=1, keepdims=True)
    alpha2 = ex2 / (den2 + 1e-16)
    out = lax.dot_general(alpha2, h2.reshape(GB, N, HID), _BAT,
                          preferred_element_type=f32)
    out_ref[...] = out + b2_ref[...]


@jax.jit
def kernel(atoms, adjs, edges, W1, att_src1, att_dst1, lin_e1, att_e1, b1,
           W2, att_src2, att_dst2, lin_e2, att_e2, b2):
    grid = (B // GB,)
    bcast = lambda shape: pl.BlockSpec(shape, lambda g: (0,) * len(shape))
    out = pl.pallas_call(
        _gat_kernel,
        grid=grid,
        in_specs=[
            pl.BlockSpec((GB, N, ATOM_DIM), lambda g: (g, 0, 0)),
            pl.BlockSpec((GB, N, N), lambda g: (g, 0, 0)),
            pl.BlockSpec((GB, N, N * EDGE_DIM), lambda g: (g, 0, 0)),
            bcast((ATOM_DIM, HEADS * HID)),
            bcast((HEADS, HID)),
            bcast((HEADS, HID)),
            bcast((EDGE_DIM, HEADS * HID)),
            bcast((HEADS, HID)),
            bcast((HEADS * HID,)),
            bcast((HEADS * HID, HID)),
            bcast((1, HID)),
            bcast((1, HID)),
            bcast((EDGE_DIM, HID)),
            bcast((1, HID)),
            bcast((HID,)),
        ],
        out_specs=pl.BlockSpec((GB, N, HID), lambda g: (g, 0, 0)),
        out_shape=jax.ShapeDtypeStruct((B, N, HID), jnp.float32),
        scratch_shapes=[
            pltpu.VMEM((N * EDGE_DIM, HEADS * N), jnp.float32),
            pltpu.VMEM((N * EDGE_DIM, N), jnp.float32),
        ],
    )(atoms, adjs, edges.reshape(B, N, N * EDGE_DIM),
      W1, att_src1, att_dst1, lin_e1, att_e1, b1,
      W2, att_src2, att_dst2, lin_e2, att_e2, b2)
    return out
